# Initial kernel scaffold; baseline (speedup 1.0000x reference)
import jax
import jax.numpy as jnp
from jax.experimental import pallas as pl


def kernel(faces, face_edges, codebooks):
    raise NotImplementedError("write your pallas kernel here")


# speedup vs baseline: 1.1536x; 1.1536x over previous
import functools

import jax
import jax.numpy as jnp
from jax.experimental import pallas as pl
from jax.experimental.pallas import tpu as pltpu

_TILE = 1024


def _rvq_kernel(x_ref, cb_ref, out_ref, loss_ref, *, n_steps, inv_count):
    x = x_ref[...]
    t, d = x.shape
    k = cb_ref.shape[1]
    residual = x
    acc = jnp.zeros_like(x)
    loss = jnp.float32(0.0)
    iota = jax.lax.broadcasted_iota(jnp.int32, (t, k), 1)
    rr = jnp.sum(residual * residual, axis=-1, keepdims=True)
    for q in range(cb_ref.shape[0]):
        cb = cb_ref[q]
        c2 = jnp.sum(cb * cb, axis=-1)[None, :]
        s = jax.lax.dot_general(
            residual, cb, (((1,), (1,)), ((), ())),
            preferred_element_type=jnp.float32,
        )
        dists = (rr - 2.0 * s) + c2
        m = jnp.min(dists, axis=-1, keepdims=True)
        idx = jnp.min(jnp.where(dists == m, iota, k), axis=-1, keepdims=True)
        onehot = (iota == idx).astype(jnp.float32)
        quant = jax.lax.dot_general(
            onehot, cb, (((1,), (0,)), ((), ())),
            preferred_element_type=jnp.float32,
            precision=jax.lax.Precision.HIGHEST,
        )
        diff = quant - residual
        loss = loss + jnp.sum(diff * diff)
        acc = acc + quant
        residual = residual - quant
        rr = jnp.sum(residual * residual, axis=-1, keepdims=True)
    out_ref[...] = acc
    i = pl.program_id(0)
    lv = jnp.reshape(loss, (1, 1))

    @pl.when(i == 0)
    def _init():
        loss_ref[...] = lv

    @pl.when(i > 0)
    def _accum():
        loss_ref[...] = loss_ref[...] + lv

    @pl.when(i == n_steps - 1)
    def _finish():
        loss_ref[...] = loss_ref[...] * inv_count


def kernel(faces, face_edges, codebooks):
    del face_edges
    b, n, d = faces.shape
    m = b * n
    flat = faces.reshape(m, d)
    n_steps = m // _TILE
    out, loss = pl.pallas_call(
        functools.partial(
            _rvq_kernel, n_steps=n_steps, inv_count=1.0 / (m * d)
        ),
        grid=(n_steps,),
        in_specs=[
            pl.BlockSpec((_TILE, d), lambda i: (i, 0)),
            pl.BlockSpec(codebooks.shape, lambda i: (0, 0, 0)),
        ],
        out_specs=[
            pl.BlockSpec((_TILE, d), lambda i: (i, 0)),
            pl.BlockSpec((1, 1), lambda i: (0, 0)),
        ],
        out_shape=[
            jax.ShapeDtypeStruct((m, d), jnp.float32),
            jax.ShapeDtypeStruct((1, 1), jnp.float32),
        ],
        compiler_params=pltpu.CompilerParams(
            dimension_semantics=("arbitrary",),
        ),
    )(flat, codebooks)
    return out.reshape(b, n, d), loss[0, 0]


# speedup vs baseline: 2.4885x; 2.1572x over previous
import functools

import jax
import jax.numpy as jnp
from jax.experimental import pallas as pl
from jax.experimental.pallas import tpu as pltpu

_TILE = 1024


def _rvq_kernel(x_ref, cb_ref, out_ref, loss_ref, *, n_steps, inv_count):
    x = x_ref[...]
    t, d = x.shape
    k = cb_ref.shape[1]
    residual = x
    acc = jnp.zeros_like(x)
    loss = jnp.float32(0.0)
    iota = jax.lax.broadcasted_iota(jnp.int32, (t, k), 1).astype(jnp.float32)
    rr = jnp.sum(residual * residual, axis=-1, keepdims=True)
    for q in range(cb_ref.shape[0]):
        cb = cb_ref[q]
        c2 = jnp.sum(cb * cb, axis=-1)[None, :]
        s = jax.lax.dot_general(
            residual, cb * -2.0, (((1,), (1,)), ((), ())),
            preferred_element_type=jnp.float32,
        )
        dists = (rr + s) + c2
        m = jnp.min(dists, axis=-1, keepdims=True)
        idx = jnp.min(
            jnp.where(dists == m, iota, jnp.float32(k)), axis=-1, keepdims=True
        )
        onehot = (iota == idx).astype(jnp.bfloat16)
        cb_hi = cb.astype(jnp.bfloat16)
        cb_lo = (cb - cb_hi.astype(jnp.float32)).astype(jnp.bfloat16)
        quant = jax.lax.dot_general(
            onehot, cb_hi, (((1,), (0,)), ((), ())),
            preferred_element_type=jnp.float32,
        ) + jax.lax.dot_general(
            onehot, cb_lo, (((1,), (0,)), ((), ())),
            preferred_element_type=jnp.float32,
        )
        diff = quant - residual
        loss = loss + jnp.sum(diff * diff)
        acc = acc + quant
        residual = residual - quant
        rr = jnp.sum(residual * residual, axis=-1, keepdims=True)
    out_ref[...] = acc
    i = pl.program_id(0)
    lv = jnp.reshape(loss, (1, 1))

    @pl.when(i == 0)
    def _init():
        loss_ref[...] = lv

    @pl.when(i > 0)
    def _accum():
        loss_ref[...] = loss_ref[...] + lv

    @pl.when(i == n_steps - 1)
    def _finish():
        loss_ref[...] = loss_ref[...] * inv_count


def kernel(faces, face_edges, codebooks):
    del face_edges
    b, n, d = faces.shape
    m = b * n
    flat = faces.reshape(m, d)
    n_steps = m // _TILE
    out, loss = pl.pallas_call(
        functools.partial(
            _rvq_kernel, n_steps=n_steps, inv_count=1.0 / (m * d)
        ),
        grid=(n_steps,),
        in_specs=[
            pl.BlockSpec((_TILE, d), lambda i: (i, 0)),
            pl.BlockSpec(codebooks.shape, lambda i: (0, 0, 0)),
        ],
        out_specs=[
            pl.BlockSpec((_TILE, d), lambda i: (i, 0)),
            pl.BlockSpec((1, 1), lambda i: (0, 0)),
        ],
        out_shape=[
            jax.ShapeDtypeStruct((m, d), jnp.float32),
            jax.ShapeDtypeStruct((1, 1), jnp.float32),
        ],
        compiler_params=pltpu.CompilerParams(
            dimension_semantics=("arbitrary",),
        ),
    )(flat, codebooks)
    return out.reshape(b, n, d), loss[0, 0]


# speedup vs baseline: 2.8907x; 1.1616x over previous
import functools

import jax
import jax.numpy as jnp
from jax.experimental import pallas as pl
from jax.experimental.pallas import tpu as pltpu

_TILE = 1024


def _rvq_kernel(x_ref, cb_ref, out_ref, loss_ref, *, n_steps, inv_count):
    x = x_ref[...]
    t, d = x.shape
    k = cb_ref.shape[1]
    residual = x
    acc = jnp.zeros_like(x)
    loss = jnp.float32(0.0)
    rr = jnp.sum(residual * residual, axis=-1, keepdims=True)
    for q in range(cb_ref.shape[0]):
        cb = cb_ref[q]
        c2 = jnp.sum(cb * cb, axis=-1)[None, :]
        s = jax.lax.dot_general(
            residual, cb * -2.0, (((1,), (1,)), ((), ())),
            preferred_element_type=jnp.float32,
        )
        dists = (rr + s) + c2
        m = jnp.min(dists, axis=-1, keepdims=True)
        onehot = (dists == m).astype(jnp.bfloat16)
        cb_hi = cb.astype(jnp.bfloat16)
        cb_lo = (cb - cb_hi.astype(jnp.float32)).astype(jnp.bfloat16)
        quant = jax.lax.dot_general(
            onehot, cb_hi, (((1,), (0,)), ((), ())),
            preferred_element_type=jnp.float32,
        ) + jax.lax.dot_general(
            onehot, cb_lo, (((1,), (0,)), ((), ())),
            preferred_element_type=jnp.float32,
        )
        loss = loss + jnp.sum(m)
        acc = acc + quant
        residual = residual - quant
        rr = jnp.sum(residual * residual, axis=-1, keepdims=True)
    out_ref[...] = acc
    i = pl.program_id(0)
    lv = jnp.reshape(loss, (1, 1))

    @pl.when(i == 0)
    def _init():
        loss_ref[...] = lv

    @pl.when(i > 0)
    def _accum():
        loss_ref[...] = loss_ref[...] + lv

    @pl.when(i == n_steps - 1)
    def _finish():
        loss_ref[...] = loss_ref[...] * inv_count


def kernel(faces, face_edges, codebooks):
    del face_edges
    b, n, d = faces.shape
    m = b * n
    flat = faces.reshape(m, d)
    n_steps = m // _TILE
    out, loss = pl.pallas_call(
        functools.partial(
            _rvq_kernel, n_steps=n_steps, inv_count=1.0 / (m * d)
        ),
        grid=(n_steps,),
        in_specs=[
            pl.BlockSpec((_TILE, d), lambda i: (i, 0)),
            pl.BlockSpec(codebooks.shape, lambda i: (0, 0, 0)),
        ],
        out_specs=[
            pl.BlockSpec((_TILE, d), lambda i: (i, 0)),
            pl.BlockSpec((1, 1), lambda i: (0, 0)),
        ],
        out_shape=[
            jax.ShapeDtypeStruct((m, d), jnp.float32),
            jax.ShapeDtypeStruct((1, 1), jnp.float32),
        ],
        compiler_params=pltpu.CompilerParams(
            dimension_semantics=("arbitrary",),
        ),
    )(flat, codebooks)
    return out.reshape(b, n, d), loss[0, 0]


# speedup vs baseline: 3.3725x; 1.1667x over previous
import functools

import jax
import jax.numpy as jnp
from jax.experimental import pallas as pl
from jax.experimental.pallas import tpu as pltpu

_TILE = 1024


def _rvq_kernel(x_ref, cb_ref, out_ref, loss_ref, c2_ref, cbg_ref, cbn_ref,
                *, n_steps, inv_count):
    nq, k, d = cb_ref.shape
    i = pl.program_id(0)

    @pl.when(i == 0)
    def _prep():
        for q in range(nq):
            cb = cb_ref[q]
            c2_ref[q] = jnp.sum(cb * cb, axis=-1)
            cb_hi = cb.astype(jnp.bfloat16)
            cb_lo = (cb - cb_hi.astype(jnp.float32)).astype(jnp.bfloat16)
            cbg_ref[q] = jnp.concatenate([cb_hi, cb_lo], axis=-1)
            cbn_ref[q] = cb * -2.0

    x = x_ref[...]
    residual = x
    acc = jnp.zeros_like(x)
    loss = jnp.float32(0.0)
    rr = jnp.sum(residual * residual, axis=-1, keepdims=True)
    for q in range(nq):
        c2 = c2_ref[q][None, :]
        s = jax.lax.dot_general(
            residual, cbn_ref[q], (((1,), (1,)), ((), ())),
            preferred_element_type=jnp.float32,
        )
        dists = (rr + s) + c2
        m = jnp.min(dists, axis=-1, keepdims=True)
        onehot = (dists == m).astype(jnp.bfloat16)
        qhl = jax.lax.dot_general(
            onehot, cbg_ref[q], (((1,), (0,)), ((), ())),
            preferred_element_type=jnp.float32,
        )
        quant = qhl[:, :d] + qhl[:, d:]
        loss = loss + jnp.sum(m)
        acc = acc + quant
        residual = residual - quant
        rr = jnp.sum(residual * residual, axis=-1, keepdims=True)
    out_ref[...] = acc
    lv = jnp.reshape(loss, (1, 1))

    @pl.when(i == 0)
    def _init():
        loss_ref[...] = lv

    @pl.when(i > 0)
    def _accum():
        loss_ref[...] = loss_ref[...] + lv

    @pl.when(i == n_steps - 1)
    def _finish():
        loss_ref[...] = loss_ref[...] * inv_count


def kernel(faces, face_edges, codebooks):
    del face_edges
    b, n, d = faces.shape
    nq, k, _ = codebooks.shape
    m = b * n
    flat = faces.reshape(m, d)
    n_steps = m // _TILE
    out, loss = pl.pallas_call(
        functools.partial(
            _rvq_kernel, n_steps=n_steps, inv_count=1.0 / (m * d)
        ),
        grid=(n_steps,),
        in_specs=[
            pl.BlockSpec((_TILE, d), lambda i: (i, 0)),
            pl.BlockSpec(codebooks.shape, lambda i: (0, 0, 0)),
        ],
        out_specs=[
            pl.BlockSpec((_TILE, d), lambda i: (i, 0)),
            pl.BlockSpec((1, 1), lambda i: (0, 0)),
        ],
        out_shape=[
            jax.ShapeDtypeStruct((m, d), jnp.float32),
            jax.ShapeDtypeStruct((1, 1), jnp.float32),
        ],
        scratch_shapes=[
            pltpu.VMEM((nq, k), jnp.float32),
            pltpu.VMEM((nq, k, 2 * d), jnp.bfloat16),
            pltpu.VMEM((nq, k, d), jnp.float32),
        ],
        compiler_params=pltpu.CompilerParams(
            dimension_semantics=("arbitrary",),
        ),
    )(flat, codebooks)
    return out.reshape(b, n, d), loss[0, 0]


# speedup vs baseline: 3.8391x; 1.1384x over previous
import functools

import jax
import jax.numpy as jnp
from jax.experimental import pallas as pl
from jax.experimental.pallas import tpu as pltpu

_TILE = 2048


def _prep_kernel(cb_ref, c2_ref, cbn_ref, cbg_ref):
    for q in range(cb_ref.shape[0]):
        cb = cb_ref[q]
        c2_ref[q] = jnp.sum(cb * cb, axis=-1)[None, :]
        cbn_ref[q] = cb * -2.0
        cb_hi = cb.astype(jnp.bfloat16)
        cb_lo = (cb - cb_hi.astype(jnp.float32)).astype(jnp.bfloat16)
        cbg_ref[q] = jnp.concatenate([cb_hi, cb_lo], axis=-1)


def _rvq_kernel(x_ref, c2_ref, cbn_ref, cbg_ref, out_ref, loss_ref,
                *, n_steps, inv_count):
    nq, k, d = cbn_ref.shape
    x = x_ref[...]
    residual = x
    acc = jnp.zeros_like(x)
    loss = jnp.float32(0.0)
    rr = jnp.sum(residual * residual, axis=-1, keepdims=True)
    for q in range(nq):
        c2 = c2_ref[q]
        s = jax.lax.dot_general(
            residual, cbn_ref[q], (((1,), (1,)), ((), ())),
            preferred_element_type=jnp.float32,
        )
        dists = (rr + s) + c2
        m = jnp.min(dists, axis=-1, keepdims=True)
        onehot = (dists == m).astype(jnp.bfloat16)
        qhl = jax.lax.dot_general(
            onehot, cbg_ref[q], (((1,), (0,)), ((), ())),
            preferred_element_type=jnp.float32,
        )
        quant = qhl[:, :d] + qhl[:, d:]
        loss = loss + jnp.sum(m)
        acc = acc + quant
        residual = residual - quant
        rr = jnp.sum(residual * residual, axis=-1, keepdims=True)
    out_ref[...] = acc
    i = pl.program_id(0)
    lv = jnp.reshape(loss, (1, 1))

    @pl.when(i == 0)
    def _init():
        loss_ref[...] = lv

    @pl.when(i > 0)
    def _accum():
        loss_ref[...] = loss_ref[...] + lv

    @pl.when(i == n_steps - 1)
    def _finish():
        loss_ref[...] = loss_ref[...] * inv_count


def kernel(faces, face_edges, codebooks):
    del face_edges
    b, n, d = faces.shape
    nq, k, _ = codebooks.shape
    m = b * n
    flat = faces.reshape(m, d)
    n_steps = m // _TILE
    c2, cbn, cbg = pl.pallas_call(
        _prep_kernel,
        out_shape=[
            jax.ShapeDtypeStruct((nq, 1, k), jnp.float32),
            jax.ShapeDtypeStruct((nq, k, d), jnp.float32),
            jax.ShapeDtypeStruct((nq, k, 2 * d), jnp.bfloat16),
        ],
    )(codebooks)
    out, loss = pl.pallas_call(
        functools.partial(
            _rvq_kernel, n_steps=n_steps, inv_count=1.0 / (m * d)
        ),
        grid=(n_steps,),
        in_specs=[
            pl.BlockSpec((_TILE, d), lambda i: (i, 0)),
            pl.BlockSpec((nq, 1, k), lambda i: (0, 0, 0)),
            pl.BlockSpec((nq, k, d), lambda i: (0, 0, 0)),
            pl.BlockSpec((nq, k, 2 * d), lambda i: (0, 0, 0)),
        ],
        out_specs=[
            pl.BlockSpec((_TILE, d), lambda i: (i, 0)),
            pl.BlockSpec((1, 1), lambda i: (0, 0)),
        ],
        out_shape=[
            jax.ShapeDtypeStruct((m, d), jnp.float32),
            jax.ShapeDtypeStruct((1, 1), jnp.float32),
        ],
        compiler_params=pltpu.CompilerParams(
            dimension_semantics=("arbitrary",),
        ),
    )(flat, c2, cbn, cbg)
    return out.reshape(b, n, d), loss[0, 0]
